# Initial kernel scaffold; baseline (speedup 1.0000x reference)
import jax
import jax.numpy as jnp
from jax.experimental import pallas as pl


def kernel(x, edge_index, W1, b1, W2, b2):
    raise NotImplementedError("write your pallas kernel here")


# speedup vs baseline: 5.3814x; 5.3814x over previous
import functools

import jax
import jax.numpy as jnp
from jax import lax
from jax.experimental import pallas as pl
from jax.experimental.pallas import tpu as pltpu
from jax.experimental.pallas import tpu_sc as plsc

N_NODES = 10000
N_EDGES = 320000
D_IN = 128
D_HID = 256
D_OUT = 128
N_HYPER = 10000

NC = 2
NS = 16
NW = NC * NS
EPW = N_EDGES // NW
CHUNK = 80
NCHUNK = EPW // CHUNK
NPAD = 10240
RPT = NPAD // NS
DRAIN = 64
CW = 16


def _fill_zero_2d(ref, rows, cols):
    per_row = cols // 16

    def body(t, c):
        ref[t // per_row, pl.ds((t % per_row) * 16, 16)] = jnp.zeros((16,), jnp.float32)
        return c

    lax.fori_loop(0, rows * per_row, body, 0)


def _make_sc_pass(with_counts):
    out_type = [jax.ShapeDtypeStruct((NC, NPAD, 128), jnp.float32)]
    scratch = [
        pltpu.VMEM_SHARED((NPAD, 128), jnp.float32),
        pltpu.VMEM((CHUNK,), jnp.int32),
        pltpu.VMEM((CHUNK,), jnp.int32),
        pltpu.VMEM((CHUNK, 128), jnp.float32),
        pltpu.VMEM((DRAIN, 128), jnp.float32),
        pltpu.SemaphoreType.DMA,
    ]
    if with_counts:
        out_type += [
            jax.ShapeDtypeStruct((NC, NPAD, CW), jnp.float32),
            jax.ShapeDtypeStruct((NC, NPAD, CW), jnp.float32),
        ]
        scratch += [
            pltpu.VMEM_SHARED((NPAD, CW), jnp.float32),
            pltpu.VMEM_SHARED((NPAD, CW), jnp.float32),
            pltpu.VMEM((DRAIN, CW), jnp.float32),
            pltpu.VMEM((CHUNK, CW), jnp.float32),
        ]

    mesh = plsc.VectorSubcoreMesh(core_axis_name="c", subcore_axis_name="s")

    def body(src_hbm, isrc_hbm, idst_hbm, out_hbm, *rest):
        if with_counts:
            (dcnt_hbm, bcnt_hbm, acc, isrc_v, idst_v, rows_v, zbuf, sem,
             dacc, bacc, cbuf, ones_v) = rest
        else:
            acc, isrc_v, idst_v, rows_v, zbuf, sem = rest
        cid = lax.axis_index("c")
        sid = lax.axis_index("s")
        wid = cid * NS + sid
        ebase = wid * EPW
        rbase = sid * RPT

        _fill_zero_2d(zbuf, DRAIN, 128)
        for s in range(RPT // DRAIN):
            pltpu.sync_copy(zbuf, acc.at[pl.ds(rbase + s * DRAIN, DRAIN)])
        if with_counts:
            _fill_zero_2d(cbuf, DRAIN, CW)
            for s in range(RPT // DRAIN):
                pltpu.sync_copy(cbuf, dacc.at[pl.ds(rbase + s * DRAIN, DRAIN)])
                pltpu.sync_copy(cbuf, bacc.at[pl.ds(rbase + s * DRAIN, DRAIN)])

            def fill_ones(t, c):
                ones_v[t, :] = jnp.where(
                    lax.iota(jnp.int32, 16) == 0, 1.0, 0.0
                ).astype(jnp.float32)
                return c

            lax.fori_loop(0, CHUNK, fill_ones, 0)
        plsc.subcore_barrier()

        def edge_body(j, c):
            base = ebase + j * CHUNK
            pltpu.sync_copy(isrc_hbm.at[pl.ds(base, CHUNK)], isrc_v)
            pltpu.sync_copy(idst_hbm.at[pl.ds(base, CHUNK)], idst_v)
            pltpu.async_copy(src_hbm.at[isrc_v], rows_v, sem).wait()
            pltpu.sync_copy(rows_v, acc.at[idst_v], add=True)
            if with_counts:
                pltpu.sync_copy(ones_v, dacc.at[isrc_v], add=True)
                pltpu.sync_copy(ones_v, bacc.at[idst_v], add=True)
            return c

        lax.fori_loop(0, NCHUNK, edge_body, 0)
        plsc.subcore_barrier()

        for s in range(RPT // DRAIN):
            r0 = rbase + s * DRAIN
            pltpu.sync_copy(acc.at[pl.ds(r0, DRAIN)], zbuf)
            pltpu.sync_copy(zbuf, out_hbm.at[cid, pl.ds(r0, DRAIN)])
        if with_counts:
            for s in range(RPT // DRAIN):
                r0 = rbase + s * DRAIN
                pltpu.sync_copy(dacc.at[pl.ds(r0, DRAIN)], cbuf)
                pltpu.sync_copy(cbuf, dcnt_hbm.at[cid, pl.ds(r0, DRAIN)])
                pltpu.sync_copy(bacc.at[pl.ds(r0, DRAIN)], cbuf)
                pltpu.sync_copy(cbuf, bcnt_hbm.at[cid, pl.ds(r0, DRAIN)])

    return pl.kernel(
        body,
        out_type=out_type if with_counts else out_type[0],
        mesh=mesh,
        scratch_types=scratch,
        compiler_params=pltpu.CompilerParams(use_tc_tiling_on_sc=False),
    )


_sc_pass_counts = _make_sc_pass(True)
_sc_pass = _make_sc_pass(False)


_ROWS_BLK = 2000
_GRID = N_NODES // _ROWS_BLK


def _inv_counts(c_ref):
    cnt = c_ref[0] + c_ref[1]
    return jnp.where(cnt > 0, 1.0 / jnp.maximum(cnt, 1.0), 0.0)


def _scale_body(p_ref, c_ref, q_ref):
    binv = _inv_counts(c_ref)
    q_ref[...] = (p_ref[0] + p_ref[1]) * binv


def _tc_scale(p, cnt):
    return pl.pallas_call(
        _scale_body,
        grid=(_GRID,),
        in_specs=[
            pl.BlockSpec((NC, _ROWS_BLK, 128), lambda i: (0, i, 0)),
            pl.BlockSpec((NC, _ROWS_BLK, 1), lambda i: (0, i, 0)),
        ],
        out_specs=pl.BlockSpec((_ROWS_BLK, 128), lambda i: (i, 0)),
        out_shape=jax.ShapeDtypeStruct((N_NODES, 128), jnp.float32),
    )(p, cnt)


def _mm_body(r_ref, c_ref, w1_ref, b1_ref, w2_ref, o_ref):
    dinv = _inv_counts(c_ref)
    s = (r_ref[0] + r_ref[1]) * dinv
    h = lax.dot_general(s, w1_ref[...], (((1,), (1,)), ((), ())),
                        preferred_element_type=jnp.float32)
    h = jnp.maximum(h + b1_ref[...], 0.0)
    o_ref[...] = lax.dot_general(h, w2_ref[...], (((1,), (1,)), ((), ())),
                                 preferred_element_type=jnp.float32)


def _tc_mm(r, cnt, W1, b1, W2):
    return pl.pallas_call(
        _mm_body,
        grid=(_GRID,),
        in_specs=[
            pl.BlockSpec((NC, _ROWS_BLK, 128), lambda i: (0, i, 0)),
            pl.BlockSpec((NC, _ROWS_BLK, 1), lambda i: (0, i, 0)),
            pl.BlockSpec((D_HID, D_IN), lambda i: (0, 0)),
            pl.BlockSpec((1, D_HID), lambda i: (0, 0)),
            pl.BlockSpec((D_OUT, D_HID), lambda i: (0, 0)),
        ],
        out_specs=pl.BlockSpec((_ROWS_BLK, D_OUT), lambda i: (i, 0)),
        out_shape=jax.ShapeDtypeStruct((N_NODES, D_OUT), jnp.float32),
    )(r, cnt, W1, b1, W2)


def _final_body(r_ref, c_ref, b2_ref, o_ref):
    dinv = _inv_counts(c_ref)
    o_ref[...] = (r_ref[0] + r_ref[1]) * dinv + b2_ref[...]


def _tc_final(r, cnt, b2):
    return pl.pallas_call(
        _final_body,
        grid=(_GRID,),
        in_specs=[
            pl.BlockSpec((NC, _ROWS_BLK, 128), lambda i: (0, i, 0)),
            pl.BlockSpec((NC, _ROWS_BLK, 1), lambda i: (0, i, 0)),
            pl.BlockSpec((1, D_OUT), lambda i: (0, 0)),
        ],
        out_specs=pl.BlockSpec((_ROWS_BLK, D_OUT), lambda i: (i, 0)),
        out_shape=jax.ShapeDtypeStruct((N_NODES, D_OUT), jnp.float32),
    )(r, cnt, b2)


@jax.jit
def kernel(x, edge_index, W1, b1, W2, b2):
    node_idx = edge_index[0]
    hyper_idx = edge_index[1]

    p1, dcnt, bcnt = _sc_pass_counts(x, node_idx, hyper_idx)
    dc = dcnt[:, :, 0:1]
    bc = bcnt[:, :, 0:1]
    q1 = _tc_scale(p1, bc)
    r1 = _sc_pass(q1, hyper_idx, node_idx)
    xw2 = _tc_mm(r1, dc, W1, b1.reshape(1, D_HID), W2)
    p2 = _sc_pass(xw2, node_idx, hyper_idx)
    q2 = _tc_scale(p2, bc)
    r2 = _sc_pass(q2, hyper_idx, node_idx)
    return _tc_final(r2, dc, b2.reshape(1, D_OUT))


# speedup vs baseline: 10.9204x; 2.0293x over previous
import functools

import jax
import jax.numpy as jnp
from jax import lax
from jax.experimental import pallas as pl
from jax.experimental.pallas import tpu as pltpu
from jax.experimental.pallas import tpu_sc as plsc

N_NODES = 10000
N_EDGES = 320000
D_IN = 128
D_HID = 256
D_OUT = 128
N_HYPER = 10000

NC = 2
NS = 16
NW = NC * NS
EPW = N_EDGES // NW
NPAD = 10240
RPT = NPAD // NS
CW = 16


def _fill_zero_2d(ref, rows, cols):
    per_row = cols // 16

    def body(t, c):
        ref[t // per_row, pl.ds((t % per_row) * 16, 16)] = jnp.zeros((16,), jnp.float32)
        return c

    lax.fori_loop(0, rows * per_row, body, 0)


def _make_sc_pass(with_counts):
    if with_counts:
        CH, NFULL, TAIL, ZR = 80, 125, 0, 32
    else:
        CH, NFULL, TAIL, ZR = 128, 78, 16, 64
    NPAIR = NFULL // 2
    LEFTOVER = NFULL % 2

    out_type = [jax.ShapeDtypeStruct((NC, NPAD, 128), jnp.float32)]
    scratch = [
        pltpu.VMEM_SHARED((NPAD, 128), jnp.float32),
        pltpu.VMEM((CH,), jnp.int32),
        pltpu.VMEM((CH,), jnp.int32),
        pltpu.VMEM((CH,), jnp.int32),
        pltpu.VMEM((CH,), jnp.int32),
        pltpu.VMEM((CH, 128), jnp.float32),
        pltpu.VMEM((CH, 128), jnp.float32),
        pltpu.VMEM((ZR, 128), jnp.float32),
        pltpu.SemaphoreType.DMA,
        pltpu.SemaphoreType.DMA,
        pltpu.SemaphoreType.DMA,
        pltpu.SemaphoreType.DMA,
    ]
    if TAIL:
        scratch += [
            pltpu.VMEM((TAIL,), jnp.int32),
            pltpu.VMEM((TAIL,), jnp.int32),
            pltpu.VMEM((TAIL, 128), jnp.float32),
        ]
    if with_counts:
        out_type += [
            jax.ShapeDtypeStruct((NC, NPAD, CW), jnp.float32),
            jax.ShapeDtypeStruct((NC, NPAD, CW), jnp.float32),
        ]
        scratch += [
            pltpu.VMEM_SHARED((NPAD, CW), jnp.float32),
            pltpu.VMEM_SHARED((NPAD, CW), jnp.float32),
            pltpu.VMEM((ZR, CW), jnp.float32),
            pltpu.VMEM((CH, CW), jnp.float32),
            pltpu.SemaphoreType.DMA,
        ]

    mesh = plsc.VectorSubcoreMesh(core_axis_name="c", subcore_axis_name="s")

    def body(src_hbm, isrc_hbm, idst_hbm, out_hbm, *rest):
        it = iter(rest)
        if with_counts:
            dcnt_hbm = next(it)
            bcnt_hbm = next(it)
        acc = next(it)
        isrc0, idst0, isrc1, idst1 = next(it), next(it), next(it), next(it)
        rows0, rows1 = next(it), next(it)
        zbuf = next(it)
        sem_i0, sem_i1, sem_g0, sem_g1 = next(it), next(it), next(it), next(it)
        if TAIL:
            isrc_t, idst_t, rows_t = next(it), next(it), next(it)
        if with_counts:
            dacc, bacc, cbuf, ones_v, sem_c = (
                next(it), next(it), next(it), next(it), next(it))

        cid = lax.axis_index("c")
        sid = lax.axis_index("s")
        wid = cid * NS + sid
        ebase = wid * EPW
        rbase = sid * RPT

        bufs = ((isrc0, idst0, rows0, sem_i0, sem_g0),
                (isrc1, idst1, rows1, sem_i1, sem_g1))

        def issue_idx(j, p):
            isb, idb, _, sem_i, _ = bufs[p]
            base = ebase + j * CH
            pltpu.async_copy(isrc_hbm.at[pl.ds(base, CH)], isb, sem_i)
            pltpu.async_copy(idst_hbm.at[pl.ds(base, CH)], idb, sem_i)

        def wait_idx(p):
            isb, idb, _, sem_i, _ = bufs[p]
            pltpu.make_async_copy(isrc_hbm.at[pl.ds(0, CH)], isb, sem_i).wait()
            pltpu.make_async_copy(idst_hbm.at[pl.ds(0, CH)], idb, sem_i).wait()

        issue_idx(0, 0)
        issue_idx(1, 1)

        _fill_zero_2d(zbuf, ZR, 128)
        for s in range(RPT // ZR):
            pltpu.sync_copy(zbuf, acc.at[pl.ds(rbase + s * ZR, ZR)])
        if with_counts:
            _fill_zero_2d(cbuf, ZR, CW)
            for s in range(RPT // ZR):
                pltpu.sync_copy(cbuf, dacc.at[pl.ds(rbase + s * ZR, ZR)])
                pltpu.sync_copy(cbuf, bacc.at[pl.ds(rbase + s * ZR, ZR)])

            def fill_ones(t, c):
                ones_v[t, :] = jnp.where(
                    lax.iota(jnp.int32, 16) == 0, 1.0, 0.0
                ).astype(jnp.float32)
                return c

            lax.fori_loop(0, CH, fill_ones, 0)
        plsc.subcore_barrier()

        def process_chunk(p, j):
            isb, idb, rb, _, sem_g = bufs[p]
            pltpu.make_async_copy(src_hbm.at[isb], rb, sem_g).wait()
            if with_counts:
                pltpu.async_copy(ones_v, dacc.at[isb], sem_c, add=True)
                pltpu.async_copy(ones_v, bacc.at[idb], sem_c, add=True)
            pltpu.sync_copy(rb, acc.at[idb], add=True)
            if with_counts:
                pltpu.make_async_copy(ones_v, dacc.at[isb], sem_c).wait()
                pltpu.make_async_copy(ones_v, bacc.at[idb], sem_c).wait()

            @pl.when(j + 2 < NFULL)
            def _():
                issue_idx(j + 2, p)

        def pair_body(k, c):
            for p in (0, 1):
                _, _, rb, _, sem_g = bufs[p]
                wait_idx(p)
                pltpu.async_copy(src_hbm.at[bufs[p][0]], rb, sem_g)
            for p in (0, 1):
                process_chunk(p, 2 * k + p)
            return c

        lax.fori_loop(0, NPAIR, pair_body, 0)

        if LEFTOVER:
            wait_idx(0)
            pltpu.async_copy(src_hbm.at[isrc0], rows0, sem_g0)
            process_chunk(0, NFULL - 1)

        if TAIL:
            base = ebase + NFULL * CH
            pltpu.sync_copy(isrc_hbm.at[pl.ds(base, TAIL)], isrc_t)
            pltpu.sync_copy(idst_hbm.at[pl.ds(base, TAIL)], idst_t)
            pltpu.async_copy(src_hbm.at[isrc_t], rows_t, sem_g0).wait()
            pltpu.sync_copy(rows_t, acc.at[idst_t], add=True)
        plsc.subcore_barrier()

        for s in range(RPT // ZR):
            r0 = rbase + s * ZR
            pltpu.sync_copy(acc.at[pl.ds(r0, ZR)], zbuf)
            pltpu.sync_copy(zbuf, out_hbm.at[cid, pl.ds(r0, ZR)])
        if with_counts:
            for s in range(RPT // ZR):
                r0 = rbase + s * ZR
                pltpu.sync_copy(dacc.at[pl.ds(r0, ZR)], cbuf)
                pltpu.sync_copy(cbuf, dcnt_hbm.at[cid, pl.ds(r0, ZR)])
                pltpu.sync_copy(bacc.at[pl.ds(r0, ZR)], cbuf)
                pltpu.sync_copy(cbuf, bcnt_hbm.at[cid, pl.ds(r0, ZR)])

    return pl.kernel(
        body,
        out_type=out_type if with_counts else out_type[0],
        mesh=mesh,
        scratch_types=scratch,
        compiler_params=pltpu.CompilerParams(use_tc_tiling_on_sc=False),
    )


_sc_pass_counts = _make_sc_pass(True)
_sc_pass = _make_sc_pass(False)


_ROWS_BLK = 2000
_GRID = N_NODES // _ROWS_BLK


def _inv_counts(c_ref):
    cnt = c_ref[0] + c_ref[1]
    return jnp.where(cnt > 0, 1.0 / jnp.maximum(cnt, 1.0), 0.0)


def _scale_body(p_ref, c_ref, q_ref):
    binv = _inv_counts(c_ref)
    q_ref[...] = (p_ref[0] + p_ref[1]) * binv


def _tc_scale(p, cnt):
    return pl.pallas_call(
        _scale_body,
        grid=(_GRID,),
        in_specs=[
            pl.BlockSpec((NC, _ROWS_BLK, 128), lambda i: (0, i, 0)),
            pl.BlockSpec((NC, _ROWS_BLK, 1), lambda i: (0, i, 0)),
        ],
        out_specs=pl.BlockSpec((_ROWS_BLK, 128), lambda i: (i, 0)),
        out_shape=jax.ShapeDtypeStruct((N_NODES, 128), jnp.float32),
    )(p, cnt)


def _mm_body(r_ref, c_ref, w1_ref, b1_ref, w2_ref, o_ref):
    dinv = _inv_counts(c_ref)
    s = (r_ref[0] + r_ref[1]) * dinv
    h = lax.dot_general(s, w1_ref[...], (((1,), (1,)), ((), ())),
                        preferred_element_type=jnp.float32)
    h = jnp.maximum(h + b1_ref[...], 0.0)
    o_ref[...] = lax.dot_general(h, w2_ref[...], (((1,), (1,)), ((), ())),
                                 preferred_element_type=jnp.float32)


def _tc_mm(r, cnt, W1, b1, W2):
    return pl.pallas_call(
        _mm_body,
        grid=(_GRID,),
        in_specs=[
            pl.BlockSpec((NC, _ROWS_BLK, 128), lambda i: (0, i, 0)),
            pl.BlockSpec((NC, _ROWS_BLK, 1), lambda i: (0, i, 0)),
            pl.BlockSpec((D_HID, D_IN), lambda i: (0, 0)),
            pl.BlockSpec((1, D_HID), lambda i: (0, 0)),
            pl.BlockSpec((D_OUT, D_HID), lambda i: (0, 0)),
        ],
        out_specs=pl.BlockSpec((_ROWS_BLK, D_OUT), lambda i: (i, 0)),
        out_shape=jax.ShapeDtypeStruct((N_NODES, D_OUT), jnp.float32),
    )(r, cnt, W1, b1, W2)


def _final_body(r_ref, c_ref, b2_ref, o_ref):
    dinv = _inv_counts(c_ref)
    o_ref[...] = (r_ref[0] + r_ref[1]) * dinv + b2_ref[...]


def _tc_final(r, cnt, b2):
    return pl.pallas_call(
        _final_body,
        grid=(_GRID,),
        in_specs=[
            pl.BlockSpec((NC, _ROWS_BLK, 128), lambda i: (0, i, 0)),
            pl.BlockSpec((NC, _ROWS_BLK, 1), lambda i: (0, i, 0)),
            pl.BlockSpec((1, D_OUT), lambda i: (0, 0)),
        ],
        out_specs=pl.BlockSpec((_ROWS_BLK, D_OUT), lambda i: (i, 0)),
        out_shape=jax.ShapeDtypeStruct((N_NODES, D_OUT), jnp.float32),
    )(r, cnt, b2)


@jax.jit
def kernel(x, edge_index, W1, b1, W2, b2):
    node_idx = edge_index[0]
    hyper_idx = edge_index[1]

    p1, dcnt, bcnt = _sc_pass_counts(x, node_idx, hyper_idx)
    dc = dcnt[:, :, 0:1]
    bc = bcnt[:, :, 0:1]
    q1 = _tc_scale(p1, bc)
    r1 = _sc_pass(q1, hyper_idx, node_idx)
    xw2 = _tc_mm(r1, dc, W1, b1.reshape(1, D_HID), W2)
    p2 = _sc_pass(xw2, node_idx, hyper_idx)
    q2 = _tc_scale(p2, bc)
    r2 = _sc_pass(q2, hyper_idx, node_idx)
    return _tc_final(r2, dc, b2.reshape(1, D_OUT))


# speedup vs baseline: 11.0986x; 1.0163x over previous
import functools

import jax
import jax.numpy as jnp
from jax import lax
from jax.experimental import pallas as pl
from jax.experimental.pallas import tpu as pltpu
from jax.experimental.pallas import tpu_sc as plsc

N_NODES = 10000
N_EDGES = 320000
D_IN = 128
D_HID = 256
D_OUT = 128
N_HYPER = 10000

NC = 2
NS = 16
NW = NC * NS
EPW = N_EDGES // NW
NPAD = 10240
RPT = NPAD // NS
CW = 16


def _fill_zero_2d(ref, rows, cols):
    per_row = cols // 16

    def body(t, c):
        ref[t // per_row, pl.ds((t % per_row) * 16, 16)] = jnp.zeros((16,), jnp.float32)
        return c

    lax.fori_loop(0, rows * per_row, body, 0)


def _make_sc_pass(with_counts):
    if with_counts:
        CH, NFULL, TAIL, ZR = 80, 125, 0, 32
    else:
        CH, NFULL, TAIL, ZR = 128, 78, 16, 64
    NPAIR = NFULL // 2
    LEFTOVER = NFULL % 2
    assert NPAIR >= 2

    out_type = [jax.ShapeDtypeStruct((NC, NPAD, 128), jnp.float32)]
    scratch = [
        pltpu.VMEM_SHARED((NPAD, 128), jnp.float32),
        pltpu.VMEM((CH,), jnp.int32),
        pltpu.VMEM((CH,), jnp.int32),
        pltpu.VMEM((CH,), jnp.int32),
        pltpu.VMEM((CH,), jnp.int32),
        pltpu.VMEM((CH,), jnp.int32),
        pltpu.VMEM((CH,), jnp.int32),
        pltpu.VMEM((CH, 128), jnp.float32),
        pltpu.VMEM((CH, 128), jnp.float32),
        pltpu.VMEM((ZR, 128), jnp.float32),
        pltpu.SemaphoreType.DMA,
        pltpu.SemaphoreType.DMA,
        pltpu.SemaphoreType.DMA,
        pltpu.SemaphoreType.DMA,
        pltpu.SemaphoreType.DMA,
        pltpu.SemaphoreType.DMA,
    ]
    if TAIL:
        scratch += [
            pltpu.VMEM((TAIL,), jnp.int32),
            pltpu.VMEM((TAIL,), jnp.int32),
            pltpu.VMEM((TAIL, 128), jnp.float32),
        ]
    if with_counts:
        out_type += [
            jax.ShapeDtypeStruct((NC, NPAD, CW), jnp.float32),
            jax.ShapeDtypeStruct((NC, NPAD, CW), jnp.float32),
        ]
        scratch += [
            pltpu.VMEM_SHARED((NPAD, CW), jnp.float32),
            pltpu.VMEM_SHARED((NPAD, CW), jnp.float32),
            pltpu.VMEM((ZR, CW), jnp.float32),
            pltpu.VMEM((CH, CW), jnp.float32),
            pltpu.VMEM((CH,), jnp.int32),
            pltpu.VMEM((CH,), jnp.int32),
            pltpu.SemaphoreType.DMA,
            pltpu.SemaphoreType.DMA,
        ]

    mesh = plsc.VectorSubcoreMesh(core_axis_name="c", subcore_axis_name="s")

    def body(src_hbm, isrc_hbm, idst_hbm, out_hbm, *rest):
        it = iter(rest)
        if with_counts:
            dcnt_hbm = next(it)
            bcnt_hbm = next(it)
        acc = next(it)
        isrc0, idst0, isrc1, idst1 = next(it), next(it), next(it), next(it)
        idsts0, idsts1 = next(it), next(it)
        rows0, rows1 = next(it), next(it)
        zbuf = next(it)
        sem_i0, sem_i1, sem_g0, sem_g1, sem_s0, sem_s1 = (
            next(it), next(it), next(it), next(it), next(it), next(it))
        if TAIL:
            isrc_t, idst_t, rows_t = next(it), next(it), next(it)
        if with_counts:
            dacc, bacc, cbuf, ones_v, isrcs0, isrcs1, sem_c0, sem_c1 = (
                next(it), next(it), next(it), next(it), next(it), next(it),
                next(it), next(it))
            isrcs = (isrcs0, isrcs1)
            sem_c = (sem_c0, sem_c1)

        cid = lax.axis_index("c")
        sid = lax.axis_index("s")
        wid = cid * NS + sid
        ebase = wid * EPW
        rbase = sid * RPT

        bufs = ((isrc0, idst0, idsts0, rows0, sem_i0, sem_g0, sem_s0),
                (isrc1, idst1, idsts1, rows1, sem_i1, sem_g1, sem_s1))

        def issue_idx(j, p):
            isb, idb = bufs[p][0], bufs[p][1]
            base = ebase + j * CH
            pltpu.async_copy(isrc_hbm.at[pl.ds(base, CH)], isb, sem_i := bufs[p][4])
            pltpu.async_copy(idst_hbm.at[pl.ds(base, CH)], idb, sem_i)

        def wait_idx(p):
            isb, idb, sem_i = bufs[p][0], bufs[p][1], bufs[p][4]
            pltpu.make_async_copy(isrc_hbm.at[pl.ds(0, CH)], isb, sem_i).wait()
            pltpu.make_async_copy(idst_hbm.at[pl.ds(0, CH)], idb, sem_i).wait()

        def issue_gather(p):
            isb, rb, sem_g = bufs[p][0], bufs[p][3], bufs[p][5]
            pltpu.async_copy(src_hbm.at[isb], rb, sem_g)

        def wait_gather(p):
            isb, rb, sem_g = bufs[p][0], bufs[p][3], bufs[p][5]
            pltpu.make_async_copy(src_hbm.at[isb], rb, sem_g).wait()

        def shadow_and_issue(p):
            isb, idb, ids, rb, _, _, sem_s = bufs[p]
            for t in range(CH // 16):
                ids[pl.ds(t * 16, 16)] = idb[pl.ds(t * 16, 16)]
            if with_counts:
                iss = isrcs[p]
                for t in range(CH // 16):
                    iss[pl.ds(t * 16, 16)] = isb[pl.ds(t * 16, 16)]
                pltpu.async_copy(ones_v, dacc.at[iss], sem_c[p], add=True)
                pltpu.async_copy(ones_v, bacc.at[ids], sem_c[p], add=True)
            pltpu.async_copy(rb, acc.at[ids], sem_s, add=True)

        def wait_scatter(p):
            ids, rb, sem_s = bufs[p][2], bufs[p][3], bufs[p][6]
            pltpu.make_async_copy(rb, acc.at[ids], sem_s).wait()
            if with_counts:
                iss = isrcs[p]
                pltpu.make_async_copy(ones_v, dacc.at[iss], sem_c[p]).wait()
                pltpu.make_async_copy(ones_v, bacc.at[ids], sem_c[p]).wait()

        issue_idx(0, 0)
        issue_idx(1, 1)

        _fill_zero_2d(zbuf, ZR, 128)
        for s in range(RPT // ZR):
            pltpu.sync_copy(zbuf, acc.at[pl.ds(rbase + s * ZR, ZR)])
        if with_counts:
            _fill_zero_2d(cbuf, ZR, CW)
            for s in range(RPT // ZR):
                pltpu.sync_copy(cbuf, dacc.at[pl.ds(rbase + s * ZR, ZR)])
                pltpu.sync_copy(cbuf, bacc.at[pl.ds(rbase + s * ZR, ZR)])

            def fill_ones(t, c):
                ones_v[t, :] = jnp.where(
                    lax.iota(jnp.int32, 16) == 0, 1.0, 0.0
                ).astype(jnp.float32)
                return c

            lax.fori_loop(0, CH, fill_ones, 0)
        plsc.subcore_barrier()

        for p in (0, 1):
            wait_idx(p)
            issue_gather(p)
        for p in (0, 1):
            wait_gather(p)
            shadow_and_issue(p)
            issue_idx(p + 2, p)

        def pair_body(k, c):
            for p in (0, 1):
                wait_scatter(p)
                wait_idx(p)
                issue_gather(p)
            for p in (0, 1):
                j = 2 * k + p
                wait_gather(p)
                shadow_and_issue(p)

                @pl.when(j + 2 < NFULL)
                def _():
                    issue_idx(j + 2, p)
            return c

        lax.fori_loop(1, NPAIR, pair_body, 0)

        if LEFTOVER:
            wait_scatter(0)
            wait_idx(0)
            issue_gather(0)
            wait_gather(0)
            shadow_and_issue(0)
            wait_scatter(0)
            wait_scatter(1)
        else:
            wait_scatter(0)
            wait_scatter(1)

        if TAIL:
            base = ebase + NFULL * CH
            pltpu.sync_copy(isrc_hbm.at[pl.ds(base, TAIL)], isrc_t)
            pltpu.sync_copy(idst_hbm.at[pl.ds(base, TAIL)], idst_t)
            pltpu.async_copy(src_hbm.at[isrc_t], rows_t, sem_g0).wait()
            pltpu.sync_copy(rows_t, acc.at[idst_t], add=True)
        plsc.subcore_barrier()

        for s in range(RPT // ZR):
            r0 = rbase + s * ZR
            pltpu.sync_copy(acc.at[pl.ds(r0, ZR)], zbuf)
            pltpu.sync_copy(zbuf, out_hbm.at[cid, pl.ds(r0, ZR)])
        if with_counts:
            for s in range(RPT // ZR):
                r0 = rbase + s * ZR
                pltpu.sync_copy(dacc.at[pl.ds(r0, ZR)], cbuf)
                pltpu.sync_copy(cbuf, dcnt_hbm.at[cid, pl.ds(r0, ZR)])
                pltpu.sync_copy(bacc.at[pl.ds(r0, ZR)], cbuf)
                pltpu.sync_copy(cbuf, bcnt_hbm.at[cid, pl.ds(r0, ZR)])

    return pl.kernel(
        body,
        out_type=out_type if with_counts else out_type[0],
        mesh=mesh,
        scratch_types=scratch,
        compiler_params=pltpu.CompilerParams(use_tc_tiling_on_sc=False),
    )


_sc_pass_counts = _make_sc_pass(True)
_sc_pass = _make_sc_pass(False)


_ROWS_BLK = 2000
_GRID = N_NODES // _ROWS_BLK


def _inv_counts(c_ref):
    cnt = c_ref[0] + c_ref[1]
    return jnp.where(cnt > 0, 1.0 / jnp.maximum(cnt, 1.0), 0.0)


def _scale_body(p_ref, c_ref, q_ref):
    binv = _inv_counts(c_ref)
    q_ref[...] = (p_ref[0] + p_ref[1]) * binv


def _tc_scale(p, cnt):
    return pl.pallas_call(
        _scale_body,
        grid=(_GRID,),
        in_specs=[
            pl.BlockSpec((NC, _ROWS_BLK, 128), lambda i: (0, i, 0)),
            pl.BlockSpec((NC, _ROWS_BLK, 1), lambda i: (0, i, 0)),
        ],
        out_specs=pl.BlockSpec((_ROWS_BLK, 128), lambda i: (i, 0)),
        out_shape=jax.ShapeDtypeStruct((N_NODES, 128), jnp.float32),
    )(p, cnt)


def _mm_body(r_ref, c_ref, w1_ref, b1_ref, w2_ref, o_ref):
    dinv = _inv_counts(c_ref)
    s = (r_ref[0] + r_ref[1]) * dinv
    h = lax.dot_general(s, w1_ref[...], (((1,), (1,)), ((), ())),
                        preferred_element_type=jnp.float32)
    h = jnp.maximum(h + b1_ref[...], 0.0)
    o_ref[...] = lax.dot_general(h, w2_ref[...], (((1,), (1,)), ((), ())),
                                 preferred_element_type=jnp.float32)


def _tc_mm(r, cnt, W1, b1, W2):
    return pl.pallas_call(
        _mm_body,
        grid=(_GRID,),
        in_specs=[
            pl.BlockSpec((NC, _ROWS_BLK, 128), lambda i: (0, i, 0)),
            pl.BlockSpec((NC, _ROWS_BLK, 1), lambda i: (0, i, 0)),
            pl.BlockSpec((D_HID, D_IN), lambda i: (0, 0)),
            pl.BlockSpec((1, D_HID), lambda i: (0, 0)),
            pl.BlockSpec((D_OUT, D_HID), lambda i: (0, 0)),
        ],
        out_specs=pl.BlockSpec((_ROWS_BLK, D_OUT), lambda i: (i, 0)),
        out_shape=jax.ShapeDtypeStruct((N_NODES, D_OUT), jnp.float32),
    )(r, cnt, W1, b1, W2)


def _final_body(r_ref, c_ref, b2_ref, o_ref):
    dinv = _inv_counts(c_ref)
    o_ref[...] = (r_ref[0] + r_ref[1]) * dinv + b2_ref[...]


def _tc_final(r, cnt, b2):
    return pl.pallas_call(
        _final_body,
        grid=(_GRID,),
        in_specs=[
            pl.BlockSpec((NC, _ROWS_BLK, 128), lambda i: (0, i, 0)),
            pl.BlockSpec((NC, _ROWS_BLK, 1), lambda i: (0, i, 0)),
            pl.BlockSpec((1, D_OUT), lambda i: (0, 0)),
        ],
        out_specs=pl.BlockSpec((_ROWS_BLK, D_OUT), lambda i: (i, 0)),
        out_shape=jax.ShapeDtypeStruct((N_NODES, D_OUT), jnp.float32),
    )(r, cnt, b2)


@jax.jit
def kernel(x, edge_index, W1, b1, W2, b2):
    node_idx = edge_index[0]
    hyper_idx = edge_index[1]

    p1, dcnt, bcnt = _sc_pass_counts(x, node_idx, hyper_idx)
    dc = dcnt[:, :, 0:1]
    bc = bcnt[:, :, 0:1]
    q1 = _tc_scale(p1, bc)
    r1 = _sc_pass(q1, hyper_idx, node_idx)
    xw2 = _tc_mm(r1, dc, W1, b1.reshape(1, D_HID), W2)
    p2 = _sc_pass(xw2, node_idx, hyper_idx)
    q2 = _tc_scale(p2, bc)
    r2 = _sc_pass(q2, hyper_idx, node_idx)
    return _tc_final(r2, dc, b2.reshape(1, D_OUT))


# speedup vs baseline: 12.8450x; 1.1574x over previous
import functools

import jax
import jax.numpy as jnp
from jax import lax
from jax.experimental import pallas as pl
from jax.experimental.pallas import tpu as pltpu
from jax.experimental.pallas import tpu_sc as plsc

N_NODES = 10000
N_EDGES = 320000
D_IN = 128
D_HID = 256
D_OUT = 128
N_HYPER = 10000

NC = 2
NS = 16
NW = NC * NS
EPW = N_EDGES // NW
NPAD = 10240
RPT = NPAD // NS
CW = 16


def _fill_zero_2d(ref, rows, cols, dtype=jnp.float32):
    lanes = 32 if dtype == jnp.bfloat16 else 16
    per_row = cols // lanes

    def body(t, c):
        ref[t // per_row, pl.ds((t % per_row) * lanes, lanes)] = jnp.zeros(
            (lanes,), dtype)
        return c

    lax.fori_loop(0, rows * per_row, body, 0)


def _make_sc_pass(with_counts):
    if with_counts:
        CH, NFULL, TAIL, ZR = 80, 125, 0, 32
    else:
        CH, NFULL, TAIL, ZR = 128, 78, 16, 64
    NPAIR = NFULL // 2
    LEFTOVER = NFULL % 2
    assert NPAIR >= 2

    out_type = [jax.ShapeDtypeStruct((NC, NPAD, 128), jnp.bfloat16)]
    scratch = [
        pltpu.VMEM_SHARED((NPAD, 128), jnp.bfloat16),
        pltpu.VMEM((CH,), jnp.int32),
        pltpu.VMEM((CH,), jnp.int32),
        pltpu.VMEM((CH,), jnp.int32),
        pltpu.VMEM((CH,), jnp.int32),
        pltpu.VMEM((CH,), jnp.int32),
        pltpu.VMEM((CH,), jnp.int32),
        pltpu.VMEM((CH, 128), jnp.bfloat16),
        pltpu.VMEM((CH, 128), jnp.bfloat16),
        pltpu.VMEM((ZR, 128), jnp.bfloat16),
        pltpu.SemaphoreType.DMA,
        pltpu.SemaphoreType.DMA,
        pltpu.SemaphoreType.DMA,
        pltpu.SemaphoreType.DMA,
        pltpu.SemaphoreType.DMA,
        pltpu.SemaphoreType.DMA,
    ]
    if TAIL:
        scratch += [
            pltpu.VMEM((TAIL,), jnp.int32),
            pltpu.VMEM((TAIL,), jnp.int32),
            pltpu.VMEM((TAIL, 128), jnp.bfloat16),
        ]
    if with_counts:
        out_type += [
            jax.ShapeDtypeStruct((NC, NPAD, CW), jnp.float32),
            jax.ShapeDtypeStruct((NC, NPAD, CW), jnp.float32),
        ]
        scratch += [
            pltpu.VMEM_SHARED((NPAD, CW), jnp.float32),
            pltpu.VMEM_SHARED((NPAD, CW), jnp.float32),
            pltpu.VMEM((ZR, CW), jnp.float32),
            pltpu.VMEM((CH, CW), jnp.float32),
            pltpu.VMEM((CH,), jnp.int32),
            pltpu.VMEM((CH,), jnp.int32),
            pltpu.SemaphoreType.DMA,
            pltpu.SemaphoreType.DMA,
        ]

    mesh = plsc.VectorSubcoreMesh(core_axis_name="c", subcore_axis_name="s")

    def body(src_hbm, isrc_hbm, idst_hbm, out_hbm, *rest):
        it = iter(rest)
        if with_counts:
            dcnt_hbm = next(it)
            bcnt_hbm = next(it)
        acc = next(it)
        isrc0, idst0, isrc1, idst1 = next(it), next(it), next(it), next(it)
        idsts0, idsts1 = next(it), next(it)
        rows0, rows1 = next(it), next(it)
        zbuf = next(it)
        sem_i0, sem_i1, sem_g0, sem_g1, sem_s0, sem_s1 = (
            next(it), next(it), next(it), next(it), next(it), next(it))
        if TAIL:
            isrc_t, idst_t, rows_t = next(it), next(it), next(it)
        if with_counts:
            dacc, bacc, cbuf, ones_v, isrcs0, isrcs1, sem_c0, sem_c1 = (
                next(it), next(it), next(it), next(it), next(it), next(it),
                next(it), next(it))
            isrcs = (isrcs0, isrcs1)
            sem_c = (sem_c0, sem_c1)

        cid = lax.axis_index("c")
        sid = lax.axis_index("s")
        wid = cid * NS + sid
        ebase = wid * EPW
        rbase = sid * RPT

        bufs = ((isrc0, idst0, idsts0, rows0, sem_i0, sem_g0, sem_s0),
                (isrc1, idst1, idsts1, rows1, sem_i1, sem_g1, sem_s1))

        def issue_idx(j, p):
            isb, idb = bufs[p][0], bufs[p][1]
            base = ebase + j * CH
            pltpu.async_copy(isrc_hbm.at[pl.ds(base, CH)], isb, sem_i := bufs[p][4])
            pltpu.async_copy(idst_hbm.at[pl.ds(base, CH)], idb, sem_i)

        def wait_idx(p):
            isb, idb, sem_i = bufs[p][0], bufs[p][1], bufs[p][4]
            pltpu.make_async_copy(isrc_hbm.at[pl.ds(0, CH)], isb, sem_i).wait()
            pltpu.make_async_copy(idst_hbm.at[pl.ds(0, CH)], idb, sem_i).wait()

        def issue_gather(p):
            isb, rb, sem_g = bufs[p][0], bufs[p][3], bufs[p][5]
            pltpu.async_copy(src_hbm.at[isb], rb, sem_g)

        def wait_gather(p):
            isb, rb, sem_g = bufs[p][0], bufs[p][3], bufs[p][5]
            pltpu.make_async_copy(src_hbm.at[isb], rb, sem_g).wait()

        def shadow_and_issue(p):
            isb, idb, ids, rb, _, _, sem_s = bufs[p]
            for t in range(CH // 16):
                ids[pl.ds(t * 16, 16)] = idb[pl.ds(t * 16, 16)]
            if with_counts:
                iss = isrcs[p]
                for t in range(CH // 16):
                    iss[pl.ds(t * 16, 16)] = isb[pl.ds(t * 16, 16)]
                pltpu.async_copy(ones_v, dacc.at[iss], sem_c[p], add=True)
                pltpu.async_copy(ones_v, bacc.at[ids], sem_c[p], add=True)
            pltpu.async_copy(rb, acc.at[ids], sem_s, add=True)

        def wait_scatter(p):
            ids, rb, sem_s = bufs[p][2], bufs[p][3], bufs[p][6]
            pltpu.make_async_copy(rb, acc.at[ids], sem_s).wait()
            if with_counts:
                iss = isrcs[p]
                pltpu.make_async_copy(ones_v, dacc.at[iss], sem_c[p]).wait()
                pltpu.make_async_copy(ones_v, bacc.at[ids], sem_c[p]).wait()

        issue_idx(0, 0)
        issue_idx(1, 1)

        _fill_zero_2d(zbuf, ZR, 128, jnp.bfloat16)
        for s in range(RPT // ZR):
            pltpu.sync_copy(zbuf, acc.at[pl.ds(rbase + s * ZR, ZR)])
        if with_counts:
            _fill_zero_2d(cbuf, ZR, CW)
            for s in range(RPT // ZR):
                pltpu.sync_copy(cbuf, dacc.at[pl.ds(rbase + s * ZR, ZR)])
                pltpu.sync_copy(cbuf, bacc.at[pl.ds(rbase + s * ZR, ZR)])

            def fill_ones(t, c):
                ones_v[t, :] = jnp.where(
                    lax.iota(jnp.int32, 16) == 0, 1.0, 0.0
                ).astype(jnp.float32)
                return c

            lax.fori_loop(0, CH, fill_ones, 0)
        plsc.subcore_barrier()

        for p in (0, 1):
            wait_idx(p)
            issue_gather(p)
        for p in (0, 1):
            wait_gather(p)
            shadow_and_issue(p)
            issue_idx(p + 2, p)

        def pair_body(k, c):
            for p in (0, 1):
                wait_scatter(p)
                wait_idx(p)
                issue_gather(p)
            for p in (0, 1):
                j = 2 * k + p
                wait_gather(p)
                shadow_and_issue(p)

                @pl.when(j + 2 < NFULL)
                def _():
                    issue_idx(j + 2, p)
            return c

        lax.fori_loop(1, NPAIR, pair_body, 0)

        if LEFTOVER:
            wait_scatter(0)
            wait_idx(0)
            issue_gather(0)
            wait_gather(0)
            shadow_and_issue(0)
            wait_scatter(0)
            wait_scatter(1)
        else:
            wait_scatter(0)
            wait_scatter(1)

        if TAIL:
            base = ebase + NFULL * CH
            pltpu.sync_copy(isrc_hbm.at[pl.ds(base, TAIL)], isrc_t)
            pltpu.sync_copy(idst_hbm.at[pl.ds(base, TAIL)], idst_t)
            pltpu.async_copy(src_hbm.at[isrc_t], rows_t, sem_g0).wait()
            pltpu.sync_copy(rows_t, acc.at[idst_t], add=True)
        plsc.subcore_barrier()

        for s in range(RPT // ZR):
            r0 = rbase + s * ZR
            pltpu.sync_copy(acc.at[pl.ds(r0, ZR)], zbuf)
            pltpu.sync_copy(zbuf, out_hbm.at[cid, pl.ds(r0, ZR)])
        if with_counts:
            for s in range(RPT // ZR):
                r0 = rbase + s * ZR
                pltpu.sync_copy(dacc.at[pl.ds(r0, ZR)], cbuf)
                pltpu.sync_copy(cbuf, dcnt_hbm.at[cid, pl.ds(r0, ZR)])
                pltpu.sync_copy(bacc.at[pl.ds(r0, ZR)], cbuf)
                pltpu.sync_copy(cbuf, bcnt_hbm.at[cid, pl.ds(r0, ZR)])

    return pl.kernel(
        body,
        out_type=out_type if with_counts else out_type[0],
        mesh=mesh,
        scratch_types=scratch,
        compiler_params=pltpu.CompilerParams(use_tc_tiling_on_sc=False),
    )


_sc_pass_counts = _make_sc_pass(True)
_sc_pass = _make_sc_pass(False)


_ROWS_BLK = 2000
_GRID = N_NODES // _ROWS_BLK


def _inv_counts(c_ref):
    cnt = c_ref[0] + c_ref[1]
    return jnp.where(cnt > 0, 1.0 / jnp.maximum(cnt, 1.0), 0.0)


def _scale_body(p_ref, c_ref, q_ref):
    binv = _inv_counts(c_ref)
    p = p_ref[0].astype(jnp.float32) + p_ref[1].astype(jnp.float32)
    q_ref[...] = (p * binv).astype(jnp.bfloat16)


def _tc_scale(p, cnt):
    return pl.pallas_call(
        _scale_body,
        grid=(_GRID,),
        in_specs=[
            pl.BlockSpec((NC, _ROWS_BLK, 128), lambda i: (0, i, 0)),
            pl.BlockSpec((NC, _ROWS_BLK, 1), lambda i: (0, i, 0)),
        ],
        out_specs=pl.BlockSpec((_ROWS_BLK, 128), lambda i: (i, 0)),
        out_shape=jax.ShapeDtypeStruct((N_NODES, 128), jnp.bfloat16),
    )(p, cnt)


def _mm_body(r_ref, c_ref, w1_ref, b1_ref, w2_ref, o_ref):
    dinv = _inv_counts(c_ref)
    s = (r_ref[0].astype(jnp.float32) + r_ref[1].astype(jnp.float32)) * dinv
    h = lax.dot_general(s, w1_ref[...], (((1,), (1,)), ((), ())),
                        preferred_element_type=jnp.float32)
    h = jnp.maximum(h + b1_ref[...], 0.0)
    o_ref[...] = lax.dot_general(h, w2_ref[...], (((1,), (1,)), ((), ())),
                                 preferred_element_type=jnp.float32
                                 ).astype(jnp.bfloat16)


def _tc_mm(r, cnt, W1, b1, W2):
    return pl.pallas_call(
        _mm_body,
        grid=(_GRID,),
        in_specs=[
            pl.BlockSpec((NC, _ROWS_BLK, 128), lambda i: (0, i, 0)),
            pl.BlockSpec((NC, _ROWS_BLK, 1), lambda i: (0, i, 0)),
            pl.BlockSpec((D_HID, D_IN), lambda i: (0, 0)),
            pl.BlockSpec((1, D_HID), lambda i: (0, 0)),
            pl.BlockSpec((D_OUT, D_HID), lambda i: (0, 0)),
        ],
        out_specs=pl.BlockSpec((_ROWS_BLK, D_OUT), lambda i: (i, 0)),
        out_shape=jax.ShapeDtypeStruct((N_NODES, D_OUT), jnp.bfloat16),
    )(r, cnt, W1, b1, W2)


def _final_body(r_ref, c_ref, b2_ref, o_ref):
    dinv = _inv_counts(c_ref)
    r = r_ref[0].astype(jnp.float32) + r_ref[1].astype(jnp.float32)
    o_ref[...] = r * dinv + b2_ref[...]


def _tc_final(r, cnt, b2):
    return pl.pallas_call(
        _final_body,
        grid=(_GRID,),
        in_specs=[
            pl.BlockSpec((NC, _ROWS_BLK, 128), lambda i: (0, i, 0)),
            pl.BlockSpec((NC, _ROWS_BLK, 1), lambda i: (0, i, 0)),
            pl.BlockSpec((1, D_OUT), lambda i: (0, 0)),
        ],
        out_specs=pl.BlockSpec((_ROWS_BLK, D_OUT), lambda i: (i, 0)),
        out_shape=jax.ShapeDtypeStruct((N_NODES, D_OUT), jnp.float32),
    )(r, cnt, b2)


@jax.jit
def kernel(x, edge_index, W1, b1, W2, b2):
    node_idx = edge_index[0]
    hyper_idx = edge_index[1]

    p1, dcnt, bcnt = _sc_pass_counts(x.astype(jnp.bfloat16), node_idx, hyper_idx)
    dc = dcnt[:, :, 0:1]
    bc = bcnt[:, :, 0:1]
    q1 = _tc_scale(p1, bc)
    r1 = _sc_pass(q1, hyper_idx, node_idx)
    xw2 = _tc_mm(r1, dc, W1, b1.reshape(1, D_HID), W2)
    p2 = _sc_pass(xw2, node_idx, hyper_idx)
    q2 = _tc_scale(p2, bc)
    r2 = _sc_pass(q2, hyper_idx, node_idx)
    return _tc_final(r2, dc, b2.reshape(1, D_OUT))
